# baseline (speedup 1.0000x reference)
import functools

import jax
import jax.numpy as jnp
from jax import lax
from jax.experimental import pallas as pl
from jax.experimental.pallas import tpu as pltpu
from jax.experimental.pallas import tpu_sc as plsc

D = 1024
EPS = 1e-05
SEQ = 2048
NC = 2
NS = 16
NW = NC * NS
N_TOK = 4 * SEQ
PER_W = N_TOK // NW
CH = 32
N_CH = PER_W // CH
LANES = 16
NVEC = D // LANES


def _lane_sum(v):
    lane = lax.iota(jnp.int32, LANES)
    for sh in (8, 4, 2, 1):
        idx = lax.bitwise_and(lane + sh, LANES - 1)
        v = v + v.at[idx].get(mode="promise_in_bounds")
    return v


def _sc_body(ids_hbm, tok_hbm, pos_hbm, gam_hbm, bet_hbm, out_hbm,
             idx_v, tok_buf, pos_buf, gam_v, bet_v, sem):
    wid = lax.axis_index("s") * NC + lax.axis_index("c")
    base = wid * PER_W
    pltpu.sync_copy(ids_hbm.at[pl.ds(base, PER_W)], idx_v)
    pltpu.sync_copy(gam_hbm, gam_v)
    pltpu.sync_copy(bet_hbm, bet_v)
    pos_base = lax.rem(base, SEQ)

    def chunk_body(c, _):
        row0 = c * CH
        pltpu.async_copy(
            tok_hbm.at[idx_v.at[pl.ds(row0, CH)]], tok_buf, sem).wait()
        pltpu.sync_copy(pos_hbm.at[pl.ds(pos_base + row0, CH)], pos_buf)

        def row_body(r, _):
            def acc_body(j, carry):
                a, a2 = carry
                sl = pl.ds(j * LANES, LANES)
                x = tok_buf[r, sl] + pos_buf[r, sl]
                tok_buf[r, sl] = x
                return a + x, a2 + x * x

            zero = jnp.zeros((LANES,), jnp.float32)
            a, a2 = lax.fori_loop(0, NVEC, acc_body, (zero, zero))
            mean_v = _lane_sum(a) * (1.0 / D)
            var_v = _lane_sum(a2) * (1.0 / D) - mean_v * mean_v
            vv = var_v + EPS
            ii = lax.bitcast_convert_type(vv, jnp.int32)
            ii = jnp.int32(0x5F3759DF) - lax.shift_right_logical(ii, 1)
            y = lax.bitcast_convert_type(ii, jnp.float32)
            half = vv * 0.5
            for _ in range(3):
                y = y * (1.5 - half * y * y)

            def norm_body(j, _):
                sl = pl.ds(j * LANES, LANES)
                x = tok_buf[r, sl]
                tok_buf[r, sl] = (x - mean_v) * y * gam_v[sl] + bet_v[sl]
                return 0

            lax.fori_loop(0, NVEC, norm_body, 0)
            return 0

        lax.fori_loop(0, CH, row_body, 0)
        pltpu.sync_copy(tok_buf, out_hbm.at[pl.ds(base + row0, CH)])
        return 0

    lax.fori_loop(0, N_CH, chunk_body, 0)


@jax.jit
def _run(ids_flat, tok_table, pos_table, ln_gamma, ln_beta):
    mesh = plsc.VectorSubcoreMesh(core_axis_name="c", subcore_axis_name="s")
    kfn = pl.kernel(
        _sc_body,
        out_type=jax.ShapeDtypeStruct((N_TOK, D), jnp.float32),
        mesh=mesh,
        scratch_types=[
            pltpu.VMEM((PER_W,), jnp.int32),
            pltpu.VMEM((CH, D), jnp.float32),
            pltpu.VMEM((CH, D), jnp.float32),
            pltpu.VMEM((D,), jnp.float32),
            pltpu.VMEM((D,), jnp.float32),
            pltpu.SemaphoreType.DMA,
        ],
    )
    return kfn(ids_flat, tok_table, pos_table, ln_gamma, ln_beta)


def kernel(input_ids, tok_table, pos_table, ln_gamma, ln_beta):
    b, s = input_ids.shape
    ids_flat = input_ids.reshape(b * s).astype(jnp.int32)
    out = _run(ids_flat, tok_table, pos_table, ln_gamma, ln_beta)
    return out.reshape(b, s, D)


# speedup vs baseline: 1.6064x; 1.6064x over previous
import functools

import jax
import jax.numpy as jnp
from jax import lax
from jax.experimental import pallas as pl
from jax.experimental.pallas import tpu as pltpu
from jax.experimental.pallas import tpu_sc as plsc

D = 1024
EPS = 1e-05
SEQ = 2048
NC = 2
NS = 16
NW = NC * NS
N_TOK = 4 * SEQ
PER_W = N_TOK // NW
CH = 16
N_CH = PER_W // CH
NB = 4
LANES = 16
NVEC = D // LANES


def _lane_sum(v):
    lane = lax.iota(jnp.int32, LANES)
    for sh in (8, 4, 2, 1):
        idx = lax.bitwise_and(lane + sh, LANES - 1)
        v = v + v.at[idx].get(mode="promise_in_bounds")
    return v


def _ln_chunk(tok_buf, pos_buf, gam_v, bet_v):
    def row_body(r, _):
        a = jnp.zeros((LANES,), jnp.float32)
        a2 = jnp.zeros((LANES,), jnp.float32)
        for j in range(NVEC):
            sl = pl.ds(j * LANES, LANES)
            x = tok_buf[r, sl] + pos_buf[r, sl]
            tok_buf[r, sl] = x
            a = a + x
            a2 = a2 + x * x
        mean_v = _lane_sum(a) * (1.0 / D)
        var_v = _lane_sum(a2) * (1.0 / D) - mean_v * mean_v
        vv = var_v + EPS
        ii = lax.bitcast_convert_type(vv, jnp.int32)
        ii = jnp.int32(0x5F3759DF) - lax.shift_right_logical(ii, 1)
        y = lax.bitcast_convert_type(ii, jnp.float32)
        half = vv * 0.5
        for _ in range(3):
            y = y * (1.5 - half * y * y)
        for j in range(NVEC):
            sl = pl.ds(j * LANES, LANES)
            x = tok_buf[r, sl]
            tok_buf[r, sl] = (x - mean_v) * y * gam_v[sl] + bet_v[sl]
        return 0

    lax.fori_loop(0, CH, row_body, 0)


def _sc_body(ids_hbm, tok_hbm, pos_hbm, gam_hbm, bet_hbm, out_hbm,
             idx_v, t0, t1, t2, t3, p0, p1, gam_v, bet_v,
             g0, g1, g2, g3, s0, s1, s2, s3, q0, q1):
    tok_bufs = (t0, t1, t2, t3)
    pos_bufs = (p0, p1)
    gsem = (g0, g1, g2, g3)
    ssem = (s0, s1, s2, s3)
    psem = (q0, q1)

    wid = lax.axis_index("s") * NC + lax.axis_index("c")
    base = wid * PER_W
    pltpu.sync_copy(ids_hbm.at[pl.ds(base, PER_W)], idx_v)
    pltpu.sync_copy(gam_hbm, gam_v)
    pltpu.sync_copy(bet_hbm, bet_v)
    pos_base = lax.rem(base, SEQ)

    def gather_tok(cc, nb):
        pltpu.async_copy(
            tok_hbm.at[idx_v.at[pl.ds(cc * CH, CH)]], tok_bufs[nb], gsem[nb])

    def copy_pos(cc, nb):
        pltpu.async_copy(
            pos_hbm.at[pl.ds(pos_base + cc * CH, CH)], pos_bufs[nb], psem[nb])

    def wait_store(cc, nb):
        pltpu.make_async_copy(
            tok_bufs[nb], out_hbm.at[pl.ds(base + cc * CH, CH)],
            ssem[nb]).wait()

    gather_tok(0, 0)
    gather_tok(1, 1)
    copy_pos(0, 0)

    def outer(c0, _):
        for b in range(NB):
            cc = c0 * NB + b
            bp = b % 2
            @pl.when(cc >= 2)
            def _():
                wait_store(cc - 2, (b + 2) % NB)

            @pl.when(cc + 2 < N_CH)
            def _():
                gather_tok(cc + 2, (b + 2) % NB)

            @pl.when(cc + 1 < N_CH)
            def _():
                copy_pos(cc + 1, (bp + 1) % 2)

            pltpu.make_async_copy(
                tok_hbm.at[idx_v.at[pl.ds(cc * CH, CH)]], tok_bufs[b],
                gsem[b]).wait()
            pltpu.make_async_copy(
                pos_hbm.at[pl.ds(pos_base + cc * CH, CH)], pos_bufs[bp],
                psem[bp]).wait()

            _ln_chunk(tok_bufs[b], pos_bufs[bp], gam_v, bet_v)

            pltpu.async_copy(
                tok_bufs[b], out_hbm.at[pl.ds(base + cc * CH, CH)], ssem[b])
        return 0

    lax.fori_loop(0, N_CH // NB, outer, 0)
    wait_store(N_CH - 2, (N_CH - 2) % NB)
    wait_store(N_CH - 1, (N_CH - 1) % NB)


@jax.jit
def _run(ids_flat, tok_table, pos_table, ln_gamma, ln_beta):
    mesh = plsc.VectorSubcoreMesh(core_axis_name="c", subcore_axis_name="s")
    kfn = pl.kernel(
        _sc_body,
        out_type=jax.ShapeDtypeStruct((N_TOK, D), jnp.float32),
        mesh=mesh,
        scratch_types=[
            pltpu.VMEM((PER_W,), jnp.int32),
            pltpu.VMEM((CH, D), jnp.float32),
            pltpu.VMEM((CH, D), jnp.float32),
            pltpu.VMEM((CH, D), jnp.float32),
            pltpu.VMEM((CH, D), jnp.float32),
            pltpu.VMEM((CH, D), jnp.float32),
            pltpu.VMEM((CH, D), jnp.float32),
            pltpu.VMEM((D,), jnp.float32),
            pltpu.VMEM((D,), jnp.float32),
        ] + [pltpu.SemaphoreType.DMA] * 10,
    )
    return kfn(ids_flat, tok_table, pos_table, ln_gamma, ln_beta)


def kernel(input_ids, tok_table, pos_table, ln_gamma, ln_beta):
    b, s = input_ids.shape
    ids_flat = input_ids.reshape(b * s).astype(jnp.int32)
    out = _run(ids_flat, tok_table, pos_table, ln_gamma, ln_beta)
    return out.reshape(b, s, D)


# speedup vs baseline: 3.2639x; 2.0319x over previous
import functools

import jax
import jax.numpy as jnp
from jax import lax
from jax.experimental import pallas as pl
from jax.experimental.pallas import tpu as pltpu
from jax.experimental.pallas import tpu_sc as plsc

D = 1024
EPS = 1e-05
SEQ = 2048
NC = 2
NS = 16
NW = NC * NS
N_TOK = 4 * SEQ
PER_W = N_TOK // NW
CH = 16
N_CH = PER_W // CH
NB = 6
PREF = 4
TBLK = 256


def _gather_body(ids_hbm, tok_hbm, out_hbm, idx_v,
                 b0, b1, b2, b3, b4, b5,
                 g0, g1, g2, g3, g4, g5,
                 s0, s1, s2, s3, s4, s5):
    bufs = (b0, b1, b2, b3, b4, b5)
    gsem = (g0, g1, g2, g3, g4, g5)
    ssem = (s0, s1, s2, s3, s4, s5)

    wid = lax.axis_index("s") * NC + lax.axis_index("c")
    base = wid * PER_W
    pltpu.sync_copy(ids_hbm.at[pl.ds(base, PER_W)], idx_v)

    def gather_tok(cc):
        nb = cc % NB
        pltpu.async_copy(
            tok_hbm.at[idx_v.at[pl.ds(cc * CH, CH)]], bufs[nb], gsem[nb])

    def wait_gather(cc):
        nb = cc % NB
        pltpu.make_async_copy(
            tok_hbm.at[idx_v.at[pl.ds(cc * CH, CH)]], bufs[nb],
            gsem[nb]).wait()

    def store_out(cc):
        nb = cc % NB
        pltpu.async_copy(
            bufs[nb], out_hbm.at[pl.ds(base + cc * CH, CH)], ssem[nb])

    def wait_store(cc):
        nb = cc % NB
        pltpu.make_async_copy(
            bufs[nb], out_hbm.at[pl.ds(base + cc * CH, CH)], ssem[nb]).wait()

    for c in range(PREF):
        gather_tok(c)
    for c in range(N_CH):
        if c >= 2:
            wait_store(c - 2)
        if c + PREF < N_CH:
            gather_tok(c + PREF)
        wait_gather(c)
        store_out(c)
    wait_store(N_CH - 2)
    wait_store(N_CH - 1)


def _ln_body(x_ref, pos_ref, gam_ref, bet_ref, o_ref):
    x = x_ref[...] + pos_ref[...]
    mean = jnp.mean(x, axis=-1, keepdims=True)
    xc = x - mean
    var = jnp.mean(xc * xc, axis=-1, keepdims=True)
    o_ref[...] = xc * lax.rsqrt(var + EPS) * gam_ref[...] + bet_ref[...]


@jax.jit
def _run(ids_flat, tok_table, pos_table, ln_gamma, ln_beta):
    mesh = plsc.VectorSubcoreMesh(core_axis_name="c", subcore_axis_name="s")
    sc_gather = pl.kernel(
        _gather_body,
        out_type=jax.ShapeDtypeStruct((N_TOK, D), jnp.float32),
        mesh=mesh,
        scratch_types=[pltpu.VMEM((PER_W,), jnp.int32)]
        + [pltpu.VMEM((CH, D), jnp.float32)] * NB
        + [pltpu.SemaphoreType.DMA] * (2 * NB),
    )
    gathered = sc_gather(ids_flat, tok_table)

    tc_ln = pl.pallas_call(
        _ln_body,
        grid=(N_TOK // TBLK,),
        in_specs=[
            pl.BlockSpec((TBLK, D), lambda i: (i, 0)),
            pl.BlockSpec((TBLK, D), lambda i: (i % (SEQ // TBLK), 0)),
            pl.BlockSpec((1, D), lambda i: (0, 0)),
            pl.BlockSpec((1, D), lambda i: (0, 0)),
        ],
        out_specs=pl.BlockSpec((TBLK, D), lambda i: (i, 0)),
        out_shape=jax.ShapeDtypeStruct((N_TOK, D), jnp.float32),
    )
    return tc_ln(gathered, pos_table,
                 ln_gamma.reshape(1, D), ln_beta.reshape(1, D))


def kernel(input_ids, tok_table, pos_table, ln_gamma, ln_beta):
    b, s = input_ids.shape
    ids_flat = input_ids.reshape(b * s).astype(jnp.int32)
    out = _run(ids_flat, tok_table, pos_table, ln_gamma, ln_beta)
    return out.reshape(b, s, D)


# speedup vs baseline: 3.6502x; 1.1183x over previous
import functools

import jax
import jax.numpy as jnp
from jax import lax
from jax.experimental import pallas as pl
from jax.experimental.pallas import tpu as pltpu
from jax.experimental.pallas import tpu_sc as plsc

D = 1024
EPS = 1e-05
SEQ = 2048
NC = 2
NS = 16
NW = NC * NS
N_TOK = 4 * SEQ
NSPLIT = 2
N_HALF = N_TOK // NSPLIT
PER_W = N_HALF // NW
CH = 16
N_CH = PER_W // CH
NB = 6
PREF = 4
TBLK = 512
NPOS = SEQ // TBLK
NBATCH_H = N_HALF // SEQ


def _gather_body(ids_hbm, tok_hbm, out_hbm, idx_v,
                 b0, b1, b2, b3, b4, b5,
                 g0, g1, g2, g3, g4, g5,
                 s0, s1, s2, s3, s4, s5):
    bufs = (b0, b1, b2, b3, b4, b5)
    gsem = (g0, g1, g2, g3, g4, g5)
    ssem = (s0, s1, s2, s3, s4, s5)

    wid = lax.axis_index("s") * NC + lax.axis_index("c")
    base = wid * PER_W
    pltpu.sync_copy(ids_hbm.at[pl.ds(base, PER_W)], idx_v)

    def gather_tok(cc):
        nb = cc % NB
        pltpu.async_copy(
            tok_hbm.at[idx_v.at[pl.ds(cc * CH, CH)]], bufs[nb], gsem[nb])

    def wait_gather(cc):
        nb = cc % NB
        pltpu.make_async_copy(
            tok_hbm.at[idx_v.at[pl.ds(cc * CH, CH)]], bufs[nb],
            gsem[nb]).wait()

    def store_out(cc):
        nb = cc % NB
        pltpu.async_copy(
            bufs[nb], out_hbm.at[pl.ds(base + cc * CH, CH)], ssem[nb])

    def wait_store(cc):
        nb = cc % NB
        pltpu.make_async_copy(
            bufs[nb], out_hbm.at[pl.ds(base + cc * CH, CH)], ssem[nb]).wait()

    for c in range(PREF):
        gather_tok(c)
    for c in range(N_CH):
        if c >= 2:
            wait_store(c - 2)
        if c + PREF < N_CH:
            gather_tok(c + PREF)
        wait_gather(c)
        store_out(c)
    wait_store(N_CH - 2)
    wait_store(N_CH - 1)


def _ln_body(x_ref, pos_ref, gam_ref, bet_ref, o_ref):
    x = x_ref[...] + pos_ref[...]
    mean = jnp.mean(x, axis=-1, keepdims=True)
    xc = x - mean
    var = jnp.mean(xc * xc, axis=-1, keepdims=True)
    o_ref[...] = xc * lax.rsqrt(var + EPS) * gam_ref[...] + bet_ref[...]


def _ln_body_alias(x_ref, pos_ref, gam_ref, bet_ref, prev_ref, o_ref):
    del prev_ref
    _ln_body(x_ref, pos_ref, gam_ref, bet_ref, o_ref)


@jax.jit
def _run(ids_flat, tok_table, pos_table, ln_gamma, ln_beta):
    mesh = plsc.VectorSubcoreMesh(core_axis_name="c", subcore_axis_name="s")
    sc_gather = pl.kernel(
        _gather_body,
        out_type=jax.ShapeDtypeStruct((N_HALF, D), jnp.float32),
        mesh=mesh,
        scratch_types=[pltpu.VMEM((PER_W,), jnp.int32)]
        + [pltpu.VMEM((CH, D), jnp.float32)] * NB
        + [pltpu.SemaphoreType.DMA] * (2 * NB),
    )
    halves = [sc_gather(ids_flat[k * N_HALF:(k + 1) * N_HALF], tok_table)
              for k in range(NSPLIT)]

    gam2 = ln_gamma.reshape(1, D)
    bet2 = ln_beta.reshape(1, D)

    tc_ln0 = pl.pallas_call(
        _ln_body,
        grid=(NPOS, NBATCH_H),
        in_specs=[
            pl.BlockSpec((TBLK, D), lambda j, b: (b * NPOS + j, 0)),
            pl.BlockSpec((TBLK, D), lambda j, b: (j, 0)),
            pl.BlockSpec((1, D), lambda j, b: (0, 0)),
            pl.BlockSpec((1, D), lambda j, b: (0, 0)),
        ],
        out_specs=pl.BlockSpec((TBLK, D), lambda j, b: (b * NPOS + j, 0)),
        out_shape=jax.ShapeDtypeStruct((N_TOK, D), jnp.float32),
    )
    y = tc_ln0(halves[0], pos_table, gam2, bet2)

    off = N_HALF // TBLK
    tc_ln1 = pl.pallas_call(
        _ln_body_alias,
        grid=(NPOS, NBATCH_H),
        in_specs=[
            pl.BlockSpec((TBLK, D), lambda j, b: (b * NPOS + j, 0)),
            pl.BlockSpec((TBLK, D), lambda j, b: (j, 0)),
            pl.BlockSpec((1, D), lambda j, b: (0, 0)),
            pl.BlockSpec((1, D), lambda j, b: (0, 0)),
            pl.BlockSpec(memory_space=pl.ANY),
        ],
        out_specs=pl.BlockSpec(
            (TBLK, D), lambda j, b: (off + b * NPOS + j, 0)),
        out_shape=jax.ShapeDtypeStruct((N_TOK, D), jnp.float32),
        input_output_aliases={4: 0},
    )
    return tc_ln1(halves[1], pos_table, gam2, bet2, y)


def kernel(input_ids, tok_table, pos_table, ln_gamma, ln_beta):
    b, s = input_ids.shape
    ids_flat = input_ids.reshape(b * s).astype(jnp.int32)
    out = _run(ids_flat, tok_table, pos_table, ln_gamma, ln_beta)
    return out.reshape(b, s, D)
